# Initial kernel scaffold; baseline (speedup 1.0000x reference)
import jax
import jax.numpy as jnp
from jax.experimental import pallas as pl


def kernel(pillar_features, coords, batch_size):
    raise NotImplementedError("write your pallas kernel here")


# speedup vs baseline: 6.1010x; 6.1010x over previous
import jax
import jax.numpy as jnp
from jax import lax
from jax.experimental import pallas as pl
from jax.experimental.pallas import tpu as pltpu
from jax.experimental.pallas import tpu_sc as plsc

NY, NX, C = 496, 432, 64
P = 100000
BS = 4

SLAB_H = 2
SLABS_PER_B = NY // SLAB_H
NBIN = 2 * SLABS_PER_B
SLABS_PER_TILE = NBIN // 16
CHK = 6256
PPAD = 16 * CHK
LPT = CHK // 16
NBIN_T = NBIN + 1
NPAD = 512
HIST_W = 16 * NPAD
BINCAP = PPAD + 8 * NBIN_T + 256
KCH = 128
WIN_X16 = NX // 16


def _iota16():
    return lax.broadcasted_iota(jnp.int32, (16,), 0)


def _sc_scatter(cb, cy, cx, feat, out, s_arr, val_arr, hist, grand_v, start_v,
                base_v, chunk_v, pid_v, pr_v, x_v, yl_v, rows_v, window,
                tot_all, histg, totg, binned, sem):
    core = lax.axis_index("c")
    tile = lax.axis_index("s")
    iota = _iota16()
    pbase = pl.multiple_of(tile * CHK, 8)

    pltpu.sync_copy(cb.at[pl.ds(pbase, CHK)], s_arr)
    pltpu.sync_copy(cy.at[pl.ds(pbase, CHK)], val_arr)
    pltpu.sync_copy(cx.at[pl.ds(pbase, CHK)], hist.at[pl.ds(0, CHK)])

    def a1_body(i, _):
        b16 = s_arr[pl.ds(i * 16, 16)]
        y16 = val_arr[pl.ds(i * 16, 16)]
        x16 = hist[pl.ds(i * 16, 16)]
        pid = pbase + i * 16 + iota
        valid = (pid < P) & ((b16 >> 1) == core)
        s16 = jnp.where(valid, (b16 & 1) * SLABS_PER_B + (y16 >> 1), NBIN)
        v16 = (pid << 10) | ((y16 & 1) << 9) | x16
        s_arr[pl.ds(i * 16, 16)] = s16
        val_arr[pl.ds(i * 16, 16)] = v16
        return 0

    lax.fori_loop(0, CHK // 16, a1_body, 0)

    def z_body(i, _):
        hist[pl.ds(i * 16, 16)] = jnp.zeros((16,), jnp.int32)
        return 0

    lax.fori_loop(0, HIST_W // 16, z_body, 0)
    ones = jnp.ones((16,), jnp.int32)

    def a2_body(v, _):
        s16 = plsc.load_gather(s_arr, [iota * LPT + v])
        plsc.addupdate_scatter(hist, [s16 * 16 + iota], ones)
        return 0

    lax.fori_loop(0, LPT, a2_body, 0)

    def tot_body(g, _):
        acc = jnp.zeros((16,), jnp.int32)
        for lane in range(16):
            acc = acc + plsc.load_gather(hist, [(g * 16 + iota) * 16 + lane])
        grand_v[pl.ds(g * 16, 16)] = acc
        return 0

    lax.fori_loop(0, NPAD // 16, tot_body, 0)

    pltpu.sync_copy(hist, histg.at[tile])
    pltpu.sync_copy(grand_v, totg.at[tile])
    plsc.subcore_barrier()

    pltpu.sync_copy(totg, tot_all)

    def grand_body(i, _):
        acc = jnp.zeros((16,), jnp.int32)
        part = jnp.zeros((16,), jnp.int32)
        for t in range(16):
            row = tot_all[t, pl.ds(i * 16, 16)]
            before = (jnp.int32(t) < tile).astype(jnp.int32)
            part = part + row * before
            acc = acc + row
        grand_v[pl.ds(i * 16, 16)] = acc
        base_v[pl.ds(i * 16, 16)] = part
        return 0

    lax.fori_loop(0, NPAD // 16, grand_body, 0)

    def scan_body(g, run):
        sz = (grand_v[pl.ds(g * 16, 16)] + 7) & ~7
        incl = plsc.cumsum(sz)
        start_v[pl.ds(g * 16, 16)] = incl - sz + run
        return run + jnp.sum(sz, axis=0)

    lax.fori_loop(0, NPAD // 16, scan_body, jnp.int32(0))

    def fold_body(g, _):
        base_v[pl.ds(g * 16, 16)] = (base_v[pl.ds(g * 16, 16)]
                                     + start_v[pl.ds(g * 16, 16)])
        return 0

    lax.fori_loop(0, NPAD // 16, fold_body, 0)

    def next_body(s, _):
        row = hist[pl.ds(s * 16, 16)]
        pref = plsc.cumsum(row) - row
        base = base_v[pl.ds(s, 16)][0]
        hist[pl.ds(s * 16, 16)] = pref + base
        return 0

    lax.fori_loop(0, NBIN_T, next_body, 0)

    def b_body(v, _):
        gidx = iota * LPT + v
        s16 = plsc.load_gather(s_arr, [gidx])
        v16 = plsc.load_gather(val_arr, [gidx])
        addr = s16 * 16 + iota
        pos = plsc.load_gather(hist, [addr])
        plsc.store_scatter(hist, [addr], pos + 1)
        chunk_v[pl.ds(0, 16)] = v16
        pltpu.sync_copy(chunk_v.at[pl.ds(0, 16)], binned.at[pos])
        return 0

    lax.fori_loop(0, LPT, b_body, 0)
    plsc.subcore_barrier()

    def wz_body(i, _):
        cidx = i // (SLAB_H * WIN_X16)
        rem = i % (SLAB_H * WIN_X16)
        window[cidx, rem // WIN_X16,
               pl.ds((rem % WIN_X16) * 16, 16)] = jnp.zeros((16,), jnp.float32)
        return 0

    lax.fori_loop(0, C * SLAB_H * WIN_X16, wz_body, 0)

    zrow = jnp.zeros((16,), jnp.float32)

    def slab_body(sl, _):
        slab = tile * SLABS_PER_TILE + sl
        b_loc = slab // SLABS_PER_B
        r = slab % SLABS_PER_B
        b = 2 * core + b_loc
        start = pl.multiple_of(start_v[pl.ds(slab, 16)][0], 8)
        n = grand_v[pl.ds(slab, 16)][0]
        nch = (n + KCH - 1) // KCH

        def make_ch_body(write_feats):
            def ch_body(jj, _):
                pltpu.sync_copy(binned.at[pl.ds(start + jj * KCH, KCH)],
                                chunk_v)
                for k in range(KCH // 16):
                    v16 = chunk_v[pl.ds(k * 16, 16)]
                    pid = jnp.minimum(jnp.maximum(v16 >> 10, 0), P - 1)
                    pid_v[pl.ds(k * 16, 16)] = pid >> 1
                    pr_v[pl.ds(k * 16, 16)] = pid & 1
                    x_v[pl.ds(k * 16, 16)] = v16 & 511
                    yl_v[pl.ds(k * 16, 16)] = (v16 >> 9) & 1
                if write_feats:
                    pltpu.async_copy(feat.at[pid_v], rows_v, sem).wait()
                m = jnp.minimum(n - jj * KCH, KCH)

                def p_body(i, _):
                    yl = jnp.full((16,), yl_v[pl.ds(i, 16)][0], jnp.int32)
                    xv = jnp.full((16,), x_v[pl.ds(i, 16)][0], jnp.int32)
                    hi = jnp.full((16,), pr_v[pl.ds(i, 16)][0], jnp.int32) > 0
                    for rr in range(4):
                        if write_feats:
                            rv = jnp.where(hi,
                                           rows_v[i, pl.ds(64 + rr * 16, 16)],
                                           rows_v[i, pl.ds(rr * 16, 16)])
                        else:
                            rv = zrow
                        plsc.store_scatter(window, [rr * 16 + iota, yl, xv],
                                           rv)
                    return 0

                lax.fori_loop(0, m, p_body, 0)
                return 0

            return ch_body

        lax.fori_loop(0, nch, make_ch_body(True), 0)
        pltpu.sync_copy(window, out.at[b, :, pl.ds(SLAB_H * r, SLAB_H), :])
        lax.fori_loop(0, nch, make_ch_body(False), 0)
        return 0

    lax.fori_loop(0, SLABS_PER_TILE, slab_body, 0)


def kernel(pillar_features, coords, batch_size):
    coords = coords.astype(jnp.int32)
    pad = PPAD - P
    cb = jnp.pad(coords[:, 0], (0, pad), constant_values=255)
    cy = jnp.pad(coords[:, 2], (0, pad))
    cx = jnp.pad(coords[:, 3], (0, pad))

    f = pl.kernel(
        _sc_scatter,
        out_type=jax.ShapeDtypeStruct((BS, C, NY, NX), jnp.float32),
        mesh=plsc.VectorSubcoreMesh(core_axis_name="c", subcore_axis_name="s"),
        compiler_params=pltpu.CompilerParams(needs_layout_passes=False),
        scratch_types=[
            pltpu.VMEM((CHK,), jnp.int32),
            pltpu.VMEM((CHK,), jnp.int32),
            pltpu.VMEM((HIST_W,), jnp.int32),
            pltpu.VMEM((NPAD,), jnp.int32),
            pltpu.VMEM((NPAD,), jnp.int32),
            pltpu.VMEM((NPAD,), jnp.int32),
            pltpu.VMEM((KCH,), jnp.int32),
            pltpu.VMEM((KCH,), jnp.int32),
            pltpu.VMEM((KCH + 16,), jnp.int32),
            pltpu.VMEM((KCH + 16,), jnp.int32),
            pltpu.VMEM((KCH + 16,), jnp.int32),
            pltpu.VMEM((KCH, 2 * C), jnp.float32),
            pltpu.VMEM((C, SLAB_H, NX), jnp.float32),
            pltpu.VMEM((16, NPAD), jnp.int32),
            pltpu.VMEM_SHARED((16, HIST_W), jnp.int32),
            pltpu.VMEM_SHARED((16, NPAD), jnp.int32),
            pltpu.VMEM_SHARED((BINCAP,), jnp.int32),
            pltpu.SemaphoreType.DMA,
        ],
    )
    return f(cb, cy, cx, pillar_features.reshape(P // 2, 2 * C))
